# baseline (speedup 1.0000x reference)
import jax
import jax.numpy as jnp
from jax.experimental import pallas as pl
from jax.experimental.pallas import tpu as pltpu

NUM_CLASSES = 1000
HIDDEN_DIM = 768
SEQ_LEN = 77
BATCH = 1024
BB = 64


def _body(species_ref, w_ref, g_ref, b_ref, o_ref):
    i = pl.program_id(0)
    rows = jnp.concatenate(
        [w_ref[pl.ds(species_ref[i * BB + r], 1), :] for r in range(BB)], axis=0
    )
    mu = jnp.mean(rows, axis=-1, keepdims=True)
    var = jnp.mean(jnp.square(rows - mu), axis=-1, keepdims=True)
    o_ref[pl.ds(i * BB, BB), :] = (
        (rows - mu) * jax.lax.rsqrt(var + 1e-5) * g_ref[...] + b_ref[...]
    )


def kernel(species, W, gamma, beta):
    species = species.astype(jnp.int32)
    grid_spec = pltpu.PrefetchScalarGridSpec(
        num_scalar_prefetch=1,
        grid=(BATCH // BB,),
        in_specs=[
            pl.BlockSpec((NUM_CLASSES, HIDDEN_DIM), lambda i, s: (0, 0)),
            pl.BlockSpec((1, HIDDEN_DIM), lambda i, s: (0, 0)),
            pl.BlockSpec((1, HIDDEN_DIM), lambda i, s: (0, 0)),
        ],
        out_specs=pl.BlockSpec(memory_space=pltpu.MemorySpace.VMEM),
    )
    emb = pl.pallas_call(
        _body,
        grid_spec=grid_spec,
        out_shape=jax.ShapeDtypeStruct((BATCH, HIDDEN_DIM), jnp.float32),
        compiler_params=pltpu.CompilerParams(
            dimension_semantics=("arbitrary",),
        ),
    )(species, W, gamma.reshape(1, HIDDEN_DIM), beta.reshape(1, HIDDEN_DIM))
    return jax.lax.broadcast_in_dim(emb, (BATCH, SEQ_LEN, HIDDEN_DIM), (0, 2))
